# Initial kernel scaffold; baseline (speedup 1.0000x reference)
import jax
import jax.numpy as jnp
from jax.experimental import pallas as pl


def kernel(h, edge_index, W, a_w):
    raise NotImplementedError("write your pallas kernel here")


# speedup vs baseline: 7.8721x; 7.8721x over previous
import jax
import jax.numpy as jnp
from jax import lax
from jax.experimental import pallas as pl
from jax.experimental.pallas import tpu as pltpu
from jax.experimental.pallas import tpu_sc as plsc

N_NODES = 10000
N_PAD = 10240
N_EDGES = 320000
D = 128
ALPHA = 0.2
NC, NS = 2, 16
HALF = N_PAD // NC
TRASH = 1024
ACC_ROWS = HALF + TRASH
CH = 128
NCH = 160
EPT = NCH * CH
E_PAD = NS * EPT
VPC = CH // 16
RPH = HALF // NS
RPZ = ACC_ROWS // NS


def _prep_body(h_ref, w_ref, a_ref, wh_ref, s_ref):
    wh = lax.dot_general(h_ref[...], w_ref[...], (((1,), (1,)), ((), ())),
                         preferred_element_type=jnp.float32)
    wh_ref[...] = wh
    a = a_ref[0, :]
    s_ref[0, :] = jnp.sum(wh * a[:D][None, :], axis=1)
    s_ref[1, :] = jnp.sum(wh * a[D:][None, :], axis=1)


def _prep(hp, W, a_w):
    BR = 1024
    return pl.pallas_call(
        _prep_body,
        grid=(N_PAD // BR,),
        in_specs=[pl.BlockSpec((BR, D), lambda i: (i, 0)),
                  pl.BlockSpec((D, D), lambda i: (0, 0)),
                  pl.BlockSpec((1, 2 * D), lambda i: (0, 0))],
        out_specs=[pl.BlockSpec((BR, D), lambda i: (i, 0)),
                   pl.BlockSpec((2, BR), lambda i: (0, i))],
        out_shape=[jax.ShapeDtypeStruct((N_PAD, D), jnp.float32),
                   jax.ShapeDtypeStruct((2, N_PAD), jnp.float32)],
    )(hp, W, a_w)


def _sc_body(wh_hbm, s_hbm, src_hbm, dst_hbm, part_hbm, den_hbm,
             s1_v, s2_v, src_v, dst_v, ex_s, rows_v, zb_v, acc_sh, den_sh,
             sem):
    cid = lax.axis_index("c")
    sid = lax.axis_index("s")
    lo = cid * HALF

    pltpu.sync_copy(s_hbm.at[0], s1_v)
    pltpu.sync_copy(s_hbm.at[1], s2_v)
    pltpu.sync_copy(src_hbm.at[sid], src_v)
    pltpu.sync_copy(dst_hbm.at[sid], dst_v)

    def zero_rows(r, _):
        for j in range(D // 16):
            rows_v[r, pl.ds(j * 16, 16)] = jnp.zeros((16,), jnp.float32)
        return 0
    lax.fori_loop(0, CH, zero_rows, 0)

    def zero_zb(i, _):
        zb_v[pl.ds(pl.multiple_of(i * 16, 16), 16)] = jnp.zeros((16,),
                                                                jnp.float32)
        return 0
    lax.fori_loop(0, RPZ // 16, zero_zb, 0)

    base_z = sid * RPZ
    for b in range(RPZ // CH):
        pltpu.sync_copy(rows_v, acc_sh.at[pl.ds(base_z + b * CH, CH)])
    pltpu.sync_copy(zb_v, den_sh.at[pl.ds(base_z, RPZ)])

    plsc.subcore_barrier()

    def chunk(c, _):
        def logits(v, _):
            sl = pl.ds(pl.multiple_of(v * 16, 16), 16)
            sv = src_v[c, sl]
            dv = dst_v[c, sl]
            e = plsc.load_gather(s1_v, [sv]) + plsc.load_gather(s2_v, [dv])
            e = jnp.where(e >= 0.0, e, ALPHA * e)
            ex_s[sl] = jnp.exp(e)
            mine = (dv >= lo) & (dv < lo + HALF)
            dst_v[c, sl] = jnp.where(mine, dv - lo,
                                     HALF + (dv & (TRASH - 1)))
            return 0
        lax.fori_loop(0, VPC, logits, 0)

        pltpu.async_copy(wh_hbm.at[src_v.at[c]], rows_v, sem).wait()

        def scale(v, _):
            exvec = ex_s[pl.ds(pl.multiple_of(v * 16, 16), 16)]
            for l in range(16):
                wgt = exvec[l]
                r = v * 16 + l
                for j in range(D // 16):
                    sl = pl.ds(j * 16, 16)
                    rows_v[r, sl] = rows_v[r, sl] * wgt
            return 0
        lax.fori_loop(0, VPC, scale, 0)

        pltpu.sync_copy(rows_v, acc_sh.at[dst_v.at[c]], add=True)
        pltpu.sync_copy(ex_s, den_sh.at[dst_v.at[c]], add=True)
        return 0
    lax.fori_loop(0, NCH, chunk, 0)

    plsc.subcore_barrier()

    out0 = cid * HALF + sid * RPH
    pltpu.sync_copy(acc_sh.at[pl.ds(sid * RPH, RPH)],
                    part_hbm.at[pl.ds(out0, RPH)])
    @pl.when(sid < NS // 2)
    def _():
        pltpu.sync_copy(den_sh.at[pl.ds(sid * (2 * RPH), 2 * RPH)],
                        den_hbm.at[0, pl.ds(cid * HALF + sid * (2 * RPH),
                                            2 * RPH)])


_sc_call = pl.kernel(
    _sc_body,
    out_type=[jax.ShapeDtypeStruct((N_PAD, D), jnp.float32),
              jax.ShapeDtypeStruct((1, N_PAD), jnp.float32)],
    mesh=plsc.VectorSubcoreMesh(core_axis_name="c", subcore_axis_name="s"),
    compiler_params=pltpu.CompilerParams(needs_layout_passes=False),
    scratch_types=[
        pltpu.VMEM((N_PAD,), jnp.float32),
        pltpu.VMEM((N_PAD,), jnp.float32),
        pltpu.VMEM((NCH, CH), jnp.int32),
        pltpu.VMEM((NCH, CH), jnp.int32),
        pltpu.VMEM((CH,), jnp.float32),
        pltpu.VMEM((CH, D), jnp.float32),
        pltpu.VMEM((RPZ,), jnp.float32),
        pltpu.VMEM_SHARED((ACC_ROWS, D), jnp.float32),
        pltpu.VMEM_SHARED((ACC_ROWS,), jnp.float32),
        pltpu.SemaphoreType.DMA,
    ],
)


def _comb_body(p_ref, d_ref, o_ref):
    o_ref[...] = p_ref[...] * (1.0 / (d_ref[0] + 1e-8))[:, None]


def _comb(part, den):
    BR = 1024
    return pl.pallas_call(
        _comb_body,
        grid=(N_PAD // BR,),
        in_specs=[pl.BlockSpec((BR, D), lambda i: (i, 0)),
                  pl.BlockSpec((1, BR), lambda i: (0, i))],
        out_specs=pl.BlockSpec((BR, D), lambda i: (i, 0)),
        out_shape=jax.ShapeDtypeStruct((N_PAD, D), jnp.float32),
    )(part, den)


@jax.jit
def kernel(h, edge_index, W, a_w):
    hp = jnp.pad(h, ((0, N_PAD - N_NODES), (0, 0)))
    wh, s = _prep(hp, W, a_w)
    npad = E_PAD - N_EDGES
    src = jnp.concatenate(
        [edge_index[0].astype(jnp.int32), jnp.zeros((npad,), jnp.int32)]
    ).reshape(NS, NCH, CH)
    dst = jnp.concatenate(
        [edge_index[1].astype(jnp.int32),
         jnp.full((npad,), N_PAD - 1, jnp.int32)]
    ).reshape(NS, NCH, CH)
    part, den = _sc_call(wh, s, src, dst)
    out = _comb(part, den)
    return out[:N_NODES]


# speedup vs baseline: 8.7340x; 1.1095x over previous
import jax
import jax.numpy as jnp
from jax import lax
from jax.experimental import pallas as pl
from jax.experimental.pallas import tpu as pltpu
from jax.experimental.pallas import tpu_sc as plsc

N_NODES = 10000
N_PAD = 10240
N_EDGES = 320000
D = 128
ALPHA = 0.2
NC, NS = 2, 16
HALF = N_PAD // NC
CH = 128
NCH = 160
EPT = NCH * CH
E_PAD = NS * EPT
VPC = CH // 16
RPH = HALF // NS
BLK = 16
NBLK = NCH // BLK
NBUF = 3
EXB = 4


def _prep_body(h_ref, w_ref, a_ref, wh_ref, s_ref):
    wh = lax.dot_general(h_ref[...], w_ref[...], (((1,), (1,)), ((), ())),
                         preferred_element_type=jnp.float32)
    wh_ref[...] = wh
    a = a_ref[0, :]
    s_ref[0, :] = jnp.sum(wh * a[:D][None, :], axis=1)
    s_ref[1, :] = jnp.sum(wh * a[D:][None, :], axis=1)


def _prep(hp, W, a_w):
    BR = 1024
    return pl.pallas_call(
        _prep_body,
        grid=(N_PAD // BR,),
        in_specs=[pl.BlockSpec((BR, D), lambda i: (i, 0)),
                  pl.BlockSpec((D, D), lambda i: (0, 0)),
                  pl.BlockSpec((1, 2 * D), lambda i: (0, 0))],
        out_specs=[pl.BlockSpec((BR, D), lambda i: (i, 0)),
                   pl.BlockSpec((2, BR), lambda i: (0, i))],
        out_shape=[jax.ShapeDtypeStruct((N_PAD, D), jnp.float32),
                   jax.ShapeDtypeStruct((2, N_PAD), jnp.float32)],
    )(hp, W, a_w)


def _sc_body(wh_hbm, s_hbm, src_hbm, dst_hbm, part_hbm, den_hbm,
             s1_v, s2_v, srcb, dstb, ex_s, rows_v, zb_v, acc_sh, den_sh,
             gs0, gs1, gs2, ss0, ss1, ss2):
    cid = lax.axis_index("c")
    sid = lax.axis_index("s")
    lo = cid * HALF
    gsems = [gs0, gs1, gs2]
    ssems = [ss0, ss1, ss2]

    pltpu.sync_copy(s_hbm.at[0], s1_v)
    pltpu.sync_copy(s_hbm.at[1], s2_v)

    def zero_rows(r, _):
        for j in range(D // 16):
            rows_v[0, r, pl.ds(j * 16, 16)] = jnp.zeros((16,), jnp.float32)
        return 0
    lax.fori_loop(0, CH, zero_rows, 0)

    def zero_zb(i, _):
        zb_v[pl.ds(pl.multiple_of(i * 16, 16), 16)] = jnp.zeros((16,),
                                                                jnp.float32)
        return 0
    lax.fori_loop(0, (2 * RPH) // 16, zero_zb, 0)

    base = sid * RPH
    pltpu.sync_copy(rows_v.at[0], acc_sh.at[pl.ds(base, CH)])
    pltpu.sync_copy(rows_v.at[0], acc_sh.at[pl.ds(base + CH, CH)])
    pltpu.sync_copy(rows_v.at[0, pl.ds(0, RPH - 2 * CH)],
                    acc_sh.at[pl.ds(base + 2 * CH, RPH - 2 * CH)])
    @pl.when(sid < NS // 2)
    def _():
        pltpu.sync_copy(zb_v, den_sh.at[pl.ds(sid * (2 * RPH), 2 * RPH)])

    plsc.subcore_barrier()

    def logits(k):
        eb = k % EXB

        def body(v, _):
            sl = pl.ds(pl.multiple_of(v * 16, 16), 16)
            sv = srcb[k, sl]
            dv = dstb[k, sl]
            e = plsc.load_gather(s1_v, [sv]) + plsc.load_gather(s2_v, [dv])
            e = jnp.where(e >= 0.0, e, ALPHA * e)
            mine = (dv >= lo) & (dv < lo + HALF)
            ex_s[eb, sl] = jnp.where(mine, jnp.exp(e), 0.0)
            dstb[k, sl] = jnp.where(mine, dv - lo, dv & (HALF - 1))
            return 0
        lax.fori_loop(0, VPC, body, 0)

    def gstart(k):
        rb = k % NBUF
        return pltpu.async_copy(wh_hbm.at[srcb.at[k]], rows_v.at[rb],
                                gsems[rb])

    def scale(k):
        rb = k % NBUF
        eb = k % EXB

        def body(v, _):
            exvec = ex_s[eb, pl.ds(pl.multiple_of(v * 16, 16), 16)]
            for l in range(16):
                wgt = exvec[l]
                r = v * 16 + l
                for j in range(D // 16):
                    sl = pl.ds(j * 16, 16)
                    rows_v[rb, r, sl] = rows_v[rb, r, sl] * wgt
            return 0
        lax.fori_loop(0, VPC, body, 0)

    def sdo(k):
        rb = k % NBUF
        eb = k % EXB
        pltpu.sync_copy(rows_v.at[rb], acc_sh.at[dstb.at[k]], add=True)
        pltpu.sync_copy(ex_s.at[eb], den_sh.at[dstb.at[k]], add=True)

    def block(blk, _):
        pltpu.sync_copy(src_hbm.at[sid, pl.ds(blk * BLK, BLK)], srcb)
        pltpu.sync_copy(dst_hbm.at[sid, pl.ds(blk * BLK, BLK)], dstb)

        g = [None] * BLK
        logits(0)
        g[0] = gstart(0)
        logits(1)
        g[1] = gstart(1)
        for k in range(BLK):
            g[k].wait()
            scale(k)
            sdo(k)
            if k + 2 < BLK:
                logits(k + 2)
                g[k + 2] = gstart(k + 2)
        return 0
    lax.fori_loop(0, NBLK, block, 0)

    plsc.subcore_barrier()

    out0 = cid * HALF + sid * RPH
    pltpu.sync_copy(acc_sh.at[pl.ds(sid * RPH, RPH)],
                    part_hbm.at[pl.ds(out0, RPH)])
    @pl.when(sid < NS // 2)
    def _():
        pltpu.sync_copy(den_sh.at[pl.ds(sid * (2 * RPH), 2 * RPH)],
                        den_hbm.at[0, pl.ds(cid * HALF + sid * (2 * RPH),
                                            2 * RPH)])


_sc_call = pl.kernel(
    _sc_body,
    out_type=[jax.ShapeDtypeStruct((N_PAD, D), jnp.float32),
              jax.ShapeDtypeStruct((1, N_PAD), jnp.float32)],
    mesh=plsc.VectorSubcoreMesh(core_axis_name="c", subcore_axis_name="s"),
    compiler_params=pltpu.CompilerParams(needs_layout_passes=False),
    scratch_types=[
        pltpu.VMEM((N_PAD,), jnp.float32),
        pltpu.VMEM((N_PAD,), jnp.float32),
        pltpu.VMEM((BLK, CH), jnp.int32),
        pltpu.VMEM((BLK, CH), jnp.int32),
        pltpu.VMEM((EXB, CH), jnp.float32),
        pltpu.VMEM((NBUF, CH, D), jnp.float32),
        pltpu.VMEM((2 * RPH,), jnp.float32),
        pltpu.VMEM_SHARED((HALF, D), jnp.float32),
        pltpu.VMEM_SHARED((HALF,), jnp.float32),
        pltpu.SemaphoreType.DMA,
        pltpu.SemaphoreType.DMA,
        pltpu.SemaphoreType.DMA,
        pltpu.SemaphoreType.DMA,
        pltpu.SemaphoreType.DMA,
        pltpu.SemaphoreType.DMA,
    ],
)


def _comb_body(p_ref, d_ref, o_ref):
    o_ref[...] = p_ref[...] * (1.0 / (d_ref[0] + 1e-8))[:, None]


def _comb(part, den):
    BR = 1024
    return pl.pallas_call(
        _comb_body,
        grid=(N_PAD // BR,),
        in_specs=[pl.BlockSpec((BR, D), lambda i: (i, 0)),
                  pl.BlockSpec((1, BR), lambda i: (0, i))],
        out_specs=pl.BlockSpec((BR, D), lambda i: (i, 0)),
        out_shape=jax.ShapeDtypeStruct((N_PAD, D), jnp.float32),
    )(part, den)


@jax.jit
def kernel(h, edge_index, W, a_w):
    hp = jnp.pad(h, ((0, N_PAD - N_NODES), (0, 0)))
    wh, s = _prep(hp, W, a_w)
    npad = E_PAD - N_EDGES
    src = jnp.concatenate(
        [edge_index[0].astype(jnp.int32), jnp.zeros((npad,), jnp.int32)]
    ).reshape(NS, NCH, CH)
    dst = jnp.concatenate(
        [edge_index[1].astype(jnp.int32),
         jnp.full((npad,), N_PAD - 1, jnp.int32)]
    ).reshape(NS, NCH, CH)
    part, den = _sc_call(wh, s, src, dst)
    out = _comb(part, den)
    return out[:N_NODES]


# speedup vs baseline: 9.0079x; 1.0314x over previous
import jax
import jax.numpy as jnp
from jax import lax
from jax.experimental import pallas as pl
from jax.experimental.pallas import tpu as pltpu
from jax.experimental.pallas import tpu_sc as plsc

N_NODES = 10000
N_PAD = 10240
N_EDGES = 320000
D = 128
ALPHA = 0.2
NC, NS = 2, 16
HALF = N_PAD // NC
CH = 128
NCH = 160
EPT = NCH * CH
E_PAD = NS * EPT
VPC = CH // 16
RPH = HALF // NS
BLK = 16
NBLK = NCH // BLK
NBUF = 3
EXB = 4


def _prep_body(h_ref, w_ref, a_ref, wh_ref, s_ref):
    wh = lax.dot_general(h_ref[...], w_ref[...], (((1,), (1,)), ((), ())),
                         preferred_element_type=jnp.float32)
    wh_ref[...] = wh
    a = a_ref[0, :]
    s_ref[0, :] = jnp.sum(wh * a[:D][None, :], axis=1)
    s_ref[1, :] = jnp.sum(wh * a[D:][None, :], axis=1)


def _prep(hp, W, a_w):
    BR = 1024
    return pl.pallas_call(
        _prep_body,
        grid=(N_PAD // BR,),
        in_specs=[pl.BlockSpec((BR, D), lambda i: (i, 0)),
                  pl.BlockSpec((D, D), lambda i: (0, 0)),
                  pl.BlockSpec((1, 2 * D), lambda i: (0, 0))],
        out_specs=[pl.BlockSpec((BR, D), lambda i: (i, 0)),
                   pl.BlockSpec((2, BR), lambda i: (0, i))],
        out_shape=[jax.ShapeDtypeStruct((N_PAD, D), jnp.float32),
                   jax.ShapeDtypeStruct((2, N_PAD), jnp.float32)],
    )(hp, W, a_w)


def _sc_body(wh_hbm, s_hbm, src_hbm, dst_hbm, part_hbm, den_hbm,
             s1_v, s2_v, srcb, dstb, ex_s, rows_v, zb_v, acc_sh, den_sh,
             gs0, gs1, gs2, ss0, ss1, ss2):
    cid = lax.axis_index("c")
    sid = lax.axis_index("s")
    lo = cid * HALF
    gsems = [gs0, gs1, gs2]
    ssems = [ss0, ss1, ss2]

    pltpu.sync_copy(s_hbm.at[0], s1_v)
    pltpu.sync_copy(s_hbm.at[1], s2_v)

    def zero_rows(r, _):
        for j in range(D // 16):
            rows_v[0, r, pl.ds(j * 16, 16)] = jnp.zeros((16,), jnp.float32)
        return 0
    lax.fori_loop(0, CH, zero_rows, 0)

    def zero_zb(i, _):
        zb_v[pl.ds(pl.multiple_of(i * 16, 16), 16)] = jnp.zeros((16,),
                                                                jnp.float32)
        return 0
    lax.fori_loop(0, (2 * RPH) // 16, zero_zb, 0)

    base = sid * RPH
    pltpu.sync_copy(rows_v.at[0], acc_sh.at[pl.ds(base, CH)])
    pltpu.sync_copy(rows_v.at[0], acc_sh.at[pl.ds(base + CH, CH)])
    pltpu.sync_copy(rows_v.at[0, pl.ds(0, RPH - 2 * CH)],
                    acc_sh.at[pl.ds(base + 2 * CH, RPH - 2 * CH)])
    @pl.when(sid < NS // 2)
    def _():
        pltpu.sync_copy(zb_v, den_sh.at[pl.ds(sid * (2 * RPH), 2 * RPH)])

    plsc.subcore_barrier()

    def logits(k):
        eb = k % EXB

        def body(v, _):
            sl = pl.ds(pl.multiple_of(v * 16, 16), 16)
            sv = srcb[k, sl]
            dv = dstb[k, sl]
            e = plsc.load_gather(s1_v, [sv]) + plsc.load_gather(s2_v, [dv])
            e = jnp.where(e >= 0.0, e, ALPHA * e)
            mine = (dv >= lo) & (dv < lo + HALF)
            ex_s[eb, sl] = jnp.where(mine, jnp.exp(e), 0.0)
            dstb[k, sl] = jnp.where(mine, dv - lo, dv & (HALF - 1))
            return 0
        lax.fori_loop(0, VPC, body, 0)

    def gstart(k):
        rb = k % NBUF
        return pltpu.async_copy(wh_hbm.at[srcb.at[k]], rows_v.at[rb],
                                gsems[rb])

    def scale(k):
        rb = k % NBUF
        eb = k % EXB

        def body(v, _):
            exvec = ex_s[eb, pl.ds(pl.multiple_of(v * 16, 16), 16)]
            for l in range(16):
                wgt = exvec[l]
                r = v * 16 + l
                for j in range(D // 16):
                    sl = pl.ds(j * 16, 16)
                    rows_v[rb, r, sl] = rows_v[rb, r, sl] * wgt
            return 0
        lax.fori_loop(0, VPC, body, 0)

    def sstart(k):
        rb = k % NBUF
        eb = k % EXB
        d1 = pltpu.async_copy(rows_v.at[rb], acc_sh.at[dstb.at[k]],
                              ss0, add=True)
        d2 = pltpu.async_copy(ex_s.at[eb], den_sh.at[dstb.at[k]],
                              ss1, add=True)
        return (d1, d2)

    def block(blk, _):
        pltpu.sync_copy(src_hbm.at[sid, pl.ds(blk * BLK, BLK)], srcb)
        pltpu.sync_copy(dst_hbm.at[sid, pl.ds(blk * BLK, BLK)], dstb)

        g = [None] * BLK
        s = [None] * BLK
        logits(0)
        g[0] = gstart(0)
        logits(1)
        g[1] = gstart(1)
        for k in range(BLK):
            g[k].wait()
            scale(k)
            if k >= 1:
                s[k - 1][0].wait()
                s[k - 1][1].wait()
            s[k] = sstart(k)
            if k + 2 < BLK:
                logits(k + 2)
                g[k + 2] = gstart(k + 2)
        s[BLK - 1][0].wait()
        s[BLK - 1][1].wait()
        return 0
    lax.fori_loop(0, NBLK, block, 0)

    plsc.subcore_barrier()

    out0 = cid * HALF + sid * RPH
    pltpu.sync_copy(acc_sh.at[pl.ds(sid * RPH, RPH)],
                    part_hbm.at[pl.ds(out0, RPH)])
    @pl.when(sid < NS // 2)
    def _():
        pltpu.sync_copy(den_sh.at[pl.ds(sid * (2 * RPH), 2 * RPH)],
                        den_hbm.at[0, pl.ds(cid * HALF + sid * (2 * RPH),
                                            2 * RPH)])


_sc_call = pl.kernel(
    _sc_body,
    out_type=[jax.ShapeDtypeStruct((N_PAD, D), jnp.float32),
              jax.ShapeDtypeStruct((1, N_PAD), jnp.float32)],
    mesh=plsc.VectorSubcoreMesh(core_axis_name="c", subcore_axis_name="s"),
    compiler_params=pltpu.CompilerParams(needs_layout_passes=False),
    scratch_types=[
        pltpu.VMEM((N_PAD,), jnp.float32),
        pltpu.VMEM((N_PAD,), jnp.float32),
        pltpu.VMEM((BLK, CH), jnp.int32),
        pltpu.VMEM((BLK, CH), jnp.int32),
        pltpu.VMEM((EXB, CH), jnp.float32),
        pltpu.VMEM((NBUF, CH, D), jnp.float32),
        pltpu.VMEM((2 * RPH,), jnp.float32),
        pltpu.VMEM_SHARED((HALF, D), jnp.float32),
        pltpu.VMEM_SHARED((HALF,), jnp.float32),
        pltpu.SemaphoreType.DMA,
        pltpu.SemaphoreType.DMA,
        pltpu.SemaphoreType.DMA,
        pltpu.SemaphoreType.DMA,
        pltpu.SemaphoreType.DMA,
        pltpu.SemaphoreType.DMA,
    ],
)


def _comb_body(p_ref, d_ref, o_ref):
    o_ref[...] = p_ref[...] * (1.0 / (d_ref[0] + 1e-8))[:, None]


def _comb(part, den):
    BR = 1024
    return pl.pallas_call(
        _comb_body,
        grid=(N_PAD // BR,),
        in_specs=[pl.BlockSpec((BR, D), lambda i: (i, 0)),
                  pl.BlockSpec((1, BR), lambda i: (0, i))],
        out_specs=pl.BlockSpec((BR, D), lambda i: (i, 0)),
        out_shape=jax.ShapeDtypeStruct((N_PAD, D), jnp.float32),
    )(part, den)


@jax.jit
def kernel(h, edge_index, W, a_w):
    hp = jnp.pad(h, ((0, N_PAD - N_NODES), (0, 0)))
    wh, s = _prep(hp, W, a_w)
    npad = E_PAD - N_EDGES
    src = jnp.concatenate(
        [edge_index[0].astype(jnp.int32), jnp.zeros((npad,), jnp.int32)]
    ).reshape(NS, NCH, CH)
    dst = jnp.concatenate(
        [edge_index[1].astype(jnp.int32),
         jnp.full((npad,), N_PAD - 1, jnp.int32)]
    ).reshape(NS, NCH, CH)
    part, den = _sc_call(wh, s, src, dst)
    out = _comb(part, den)
    return out[:N_NODES]


# speedup vs baseline: 10.9874x; 1.2197x over previous
import jax
import jax.numpy as jnp
from jax import lax
from jax.experimental import pallas as pl
from jax.experimental.pallas import tpu as pltpu
from jax.experimental.pallas import tpu_sc as plsc

N_NODES = 10000
N_PAD = 10240
N_EDGES = 320000
D = 128
ALPHA = 0.2
NC, NS = 2, 16
CH = 64
NCH = 160
EPT = NCH * CH
E_PAD = NC * NS * EPT
VPC = CH // 16
RPH = N_PAD // NS
BLK = 16
NBLK = NCH // BLK
NBUF = 2
EXB = 4


def _prep_body(h_ref, w_ref, a_ref, wh_ref, s_ref):
    wh = lax.dot_general(h_ref[...], w_ref[...], (((1,), (1,)), ((), ())),
                         preferred_element_type=jnp.float32)
    wh_ref[...] = wh
    a = a_ref[0, :]
    s_ref[0, :] = jnp.sum(wh * a[:D][None, :], axis=1)
    s_ref[1, :] = jnp.sum(wh * a[D:][None, :], axis=1)


def _prep(hp, W, a_w):
    BR = 1024
    return pl.pallas_call(
        _prep_body,
        grid=(N_PAD // BR,),
        in_specs=[pl.BlockSpec((BR, D), lambda i: (i, 0)),
                  pl.BlockSpec((D, D), lambda i: (0, 0)),
                  pl.BlockSpec((1, 2 * D), lambda i: (0, 0))],
        out_specs=[pl.BlockSpec((BR, D), lambda i: (i, 0)),
                   pl.BlockSpec((2, BR), lambda i: (0, i))],
        out_shape=[jax.ShapeDtypeStruct((N_PAD, D), jnp.float32),
                   jax.ShapeDtypeStruct((2, N_PAD), jnp.float32)],
    )(hp, W, a_w)


def _sc_body(wh_hbm, s_hbm, src_hbm, dst_hbm, part_hbm, den_hbm,
             s1_v, s2_v, srcb, dstb, ex_s, rows_v, zb_v, acc_sh, den_sh,
             gs0, gs1, ss0, ss1):
    cid = lax.axis_index("c")
    sid = lax.axis_index("s")
    gsems = [gs0, gs1]

    pltpu.sync_copy(s_hbm.at[0], s1_v)
    pltpu.sync_copy(s_hbm.at[1], s2_v)

    def zero_rows(r, _):
        for j in range(D // 16):
            rows_v[0, r, pl.ds(j * 16, 16)] = jnp.zeros((16,), jnp.float32)
        return 0
    lax.fori_loop(0, CH, zero_rows, 0)

    def zero_zb(i, _):
        zb_v[pl.ds(pl.multiple_of(i * 16, 16), 16)] = jnp.zeros((16,),
                                                                jnp.float32)
        return 0
    lax.fori_loop(0, RPH // 16, zero_zb, 0)

    base = sid * RPH
    for t in range(RPH // CH):
        pltpu.sync_copy(rows_v.at[0], acc_sh.at[pl.ds(base + t * CH, CH)])
    pltpu.sync_copy(zb_v, den_sh.at[pl.ds(base, RPH)])

    plsc.subcore_barrier()

    def logits(k):
        eb = k % EXB

        def body(v, _):
            sl = pl.ds(pl.multiple_of(v * 16, 16), 16)
            sv = srcb[k, sl]
            dv = dstb[k, sl]
            e = plsc.load_gather(s1_v, [sv]) + plsc.load_gather(s2_v, [dv])
            e = jnp.where(e >= 0.0, e, ALPHA * e)
            ex_s[eb, sl] = jnp.exp(e)
            return 0
        lax.fori_loop(0, VPC, body, 0)

    def gstart(k):
        rb = k % NBUF
        return pltpu.async_copy(wh_hbm.at[srcb.at[k]], rows_v.at[rb],
                                gsems[rb])

    def scale(k):
        rb = k % NBUF
        eb = k % EXB

        def body(v, _):
            exvec = ex_s[eb, pl.ds(pl.multiple_of(v * 16, 16), 16)]
            for l in range(16):
                wgt = exvec[l]
                r = v * 16 + l
                for j in range(D // 16):
                    sl = pl.ds(j * 16, 16)
                    rows_v[rb, r, sl] = rows_v[rb, r, sl] * wgt
            return 0
        lax.fori_loop(0, VPC, body, 0)

    def sstart(k):
        rb = k % NBUF
        eb = k % EXB
        d1 = pltpu.async_copy(rows_v.at[rb], acc_sh.at[dstb.at[k]],
                              ss0, add=True)
        d2 = pltpu.async_copy(ex_s.at[eb], den_sh.at[dstb.at[k]],
                              ss1, add=True)
        return (d1, d2)

    def block(blk, _):
        pltpu.sync_copy(src_hbm.at[cid, sid, pl.ds(blk * BLK, BLK)], srcb)
        pltpu.sync_copy(dst_hbm.at[cid, sid, pl.ds(blk * BLK, BLK)], dstb)

        g = [None] * BLK
        s = [None] * BLK
        logits(0)
        g[0] = gstart(0)
        for k in range(BLK):
            g[k].wait()
            scale(k)
            if k >= 1:
                s[k - 1][0].wait()
                s[k - 1][1].wait()
            s[k] = sstart(k)
            if k + 1 < BLK:
                logits(k + 1)
                g[k + 1] = gstart(k + 1)
        s[BLK - 1][0].wait()
        s[BLK - 1][1].wait()
        return 0
    lax.fori_loop(0, NBLK, block, 0)

    plsc.subcore_barrier()

    pltpu.sync_copy(acc_sh.at[pl.ds(sid * RPH, RPH)],
                    part_hbm.at[cid, pl.ds(sid * RPH, RPH)])
    pltpu.sync_copy(den_sh.at[pl.ds(sid * RPH, RPH)],
                    den_hbm.at[cid, pl.ds(sid * RPH, RPH)])


_sc_call = pl.kernel(
    _sc_body,
    out_type=[jax.ShapeDtypeStruct((NC, N_PAD, D), jnp.float32),
              jax.ShapeDtypeStruct((NC, N_PAD), jnp.float32)],
    mesh=plsc.VectorSubcoreMesh(core_axis_name="c", subcore_axis_name="s"),
    compiler_params=pltpu.CompilerParams(needs_layout_passes=False),
    scratch_types=[
        pltpu.VMEM((N_PAD,), jnp.float32),
        pltpu.VMEM((N_PAD,), jnp.float32),
        pltpu.VMEM((BLK, CH), jnp.int32),
        pltpu.VMEM((BLK, CH), jnp.int32),
        pltpu.VMEM((EXB, CH), jnp.float32),
        pltpu.VMEM((NBUF, CH, D), jnp.float32),
        pltpu.VMEM((RPH,), jnp.float32),
        pltpu.VMEM_SHARED((N_PAD, D), jnp.float32),
        pltpu.VMEM_SHARED((N_PAD,), jnp.float32),
        pltpu.SemaphoreType.DMA,
        pltpu.SemaphoreType.DMA,
        pltpu.SemaphoreType.DMA,
        pltpu.SemaphoreType.DMA,
    ],
)


def _comb_body(p_ref, d_ref, o_ref):
    den = d_ref[0] + d_ref[1]
    o_ref[...] = (p_ref[0] + p_ref[1]) * (1.0 / (den + 1e-8))[:, None]


def _comb(part, den):
    BR = 1024
    return pl.pallas_call(
        _comb_body,
        grid=(N_PAD // BR,),
        in_specs=[pl.BlockSpec((NC, BR, D), lambda i: (0, i, 0)),
                  pl.BlockSpec((NC, BR), lambda i: (0, i))],
        out_specs=pl.BlockSpec((BR, D), lambda i: (i, 0)),
        out_shape=jax.ShapeDtypeStruct((N_PAD, D), jnp.float32),
    )(part, den)


@jax.jit
def kernel(h, edge_index, W, a_w):
    hp = jnp.pad(h, ((0, N_PAD - N_NODES), (0, 0)))
    wh, s = _prep(hp, W, a_w)
    npad = E_PAD - N_EDGES
    src = jnp.concatenate(
        [edge_index[0].astype(jnp.int32), jnp.zeros((npad,), jnp.int32)]
    ).reshape(NC, NS, NCH, CH)
    dst = jnp.concatenate(
        [edge_index[1].astype(jnp.int32),
         jnp.full((npad,), N_PAD - 1, jnp.int32)]
    ).reshape(NC, NS, NCH, CH)
    part, den = _sc_call(wh, s, src, dst)
    out = _comb(part, den)
    return out[:N_NODES]
